# baseline (speedup 1.0000x reference)
import functools

import numpy as np
import jax
import jax.numpy as jnp
from jax import lax
from jax.experimental import pallas as pl
from jax.experimental.pallas import tpu as pltpu
from jax.experimental.pallas import tpu_sc as plsc

_PS = 16
_H = 512
_W = 512
_HP = _H // _PS
_WP = _W // _PS
_TOTAL = _HP * _WP
_NUM_MASK = int(0.75 * _TOTAL)
_MASK_VALUE = 0.5
_BLOCK_ROWS = 4096

_MASK_BITS_HEX = (
    "bfbe67fd4f3fa775bcfdfe7dffefe7bbf0f9ff37fadbfefe6c7bfffaff4b5b6f"
    "fdabf03bd7ffbd7ffdeffa7f5bbe7fefe8e74efffffff7feeefffbf7f5f3b57d"
    "f9baefd79ff8febdf7f1affaceed6bb4fdcfdc3e677fbcbb4fbbf4cad97fb7ef"
    "efffffd49e3ecffdff9fe299ff5b5e9f0a65d66b75effbeefd76bdefe3dfeffd"
)


def _full_mask() -> np.ndarray:
    val = int(_MASK_BITS_HEX, 16)
    patch_mask = np.array([(val >> i) & 1 for i in range(_TOTAL)], dtype=bool)
    grid2d = patch_mask.reshape(_HP, _WP)
    return np.repeat(np.repeat(grid2d, _PS, axis=0), _PS, axis=1)


_MASK_BLOCK = np.tile(_full_mask(), (_BLOCK_ROWS // _H, 1)).astype(np.float32)


def _select_body(m_ref, x_ref, o_ref):
    o_ref[...] = jnp.where(m_ref[...] != 0.0, _MASK_VALUE, x_ref[...])


def _kernel_tc(x):
    B, C, H, W = x.shape
    rows = B * C * H
    xr = x.reshape(rows, W)
    mask = jnp.asarray(_MASK_BLOCK)
    out = pl.pallas_call(
        _select_body,
        grid=(rows // _BLOCK_ROWS,),
        in_specs=[
            pl.BlockSpec((_BLOCK_ROWS, W), lambda i: (0, 0)),
            pl.BlockSpec((_BLOCK_ROWS, W), lambda i: (i, 0)),
        ],
        out_specs=pl.BlockSpec((_BLOCK_ROWS, W), lambda i: (i, 0)),
        out_shape=jax.ShapeDtypeStruct((rows, W), x.dtype),
        compiler_params=pltpu.CompilerParams(
            dimension_semantics=("parallel",),
        ),
    )(mask, xr)
    return out.reshape(B, C, H, W)


_L = 16
_NW = 32
_ROWS = 64 * 3 * _H
_RPW = _ROWS // _NW
_CHUNK = 2 * _PS
_NCHUNK = _RPW // _CHUNK
_NBUF = 2


def _col_words() -> list:
    val = int(_MASK_BITS_HEX, 16)
    bits = [(val >> i) & 1 for i in range(_TOTAL)]
    return [sum(bits[pr * _WP + c] << pr for pr in range(_HP))
            for c in range(_WP)]


_COL_WORDS = _col_words()


def _mask_row_table() -> np.ndarray:
    full = _full_mask()
    return full[::_PS, :].astype(np.float32)


_MASK_ROWS = _mask_row_table()


def _sc_body(x_hbm, mrow_hbm, out_hbm, ibufs, obufs, isems, osems, mtab):
    wid = lax.axis_index("s") * 2 + lax.axis_index("c")
    base = wid * _NCHUNK
    pltpu.sync_copy(mrow_hbm, mtab)

    def in_slice(ci):
        return x_hbm.at[pl.ds((base + ci) * _CHUNK, _CHUNK)]

    def out_slice(ci):
        return out_hbm.at[pl.ds((base + ci) * _CHUNK, _CHUNK)]

    for b in range(_NBUF):
        pltpu.async_copy(in_slice(b), ibufs[b], isems[b])

    fill = jnp.full((_L,), _MASK_VALUE, jnp.float32)

    def round_body(gp, carry):
        for b in range(_NBUF):
            ci = gp * _NBUF + b
            pltpu.make_async_copy(in_slice(ci), ibufs[b], isems[b]).wait()
            for half in range(2):
                pr = lax.rem((base + ci) * 2 + half, _HP)
                for c in range(_W // _L):
                    sl = pl.ds(c * _L, _L)
                    mval = mtab[pr, sl]
                    for r in range(_PS):
                        rr = half * _PS + r
                        obufs[b][rr, sl] = jnp.where(
                            mval != 0.0, fill, ibufs[b][rr, sl])
            @pl.when(gp > 0)
            def _():
                pltpu.make_async_copy(
                    obufs[b], out_slice(ci - _NBUF), osems[b]).wait()
            pltpu.async_copy(obufs[b], out_slice(ci), osems[b])
            @pl.when(ci + _NBUF < _NCHUNK)
            def _():
                pltpu.async_copy(in_slice(ci + _NBUF), ibufs[b], isems[b])
        return carry

    lax.fori_loop(0, _NCHUNK // _NBUF, round_body, 0)

    for b in range(_NBUF):
        pltpu.make_async_copy(
            obufs[b], out_slice(_NCHUNK - _NBUF + b), osems[b]).wait()


def _kernel_sc(x):
    B, C, H, W = x.shape
    xr = x.reshape(B * C * H, W)
    mesh = plsc.VectorSubcoreMesh(core_axis_name="c", subcore_axis_name="s")
    run = functools.partial(
        pl.kernel,
        mesh=mesh,
        out_type=jax.ShapeDtypeStruct((B * C * H, W), x.dtype),
        scratch_types=[
            [pltpu.VMEM((_CHUNK, W), jnp.float32) for _ in range(_NBUF)],
            [pltpu.VMEM((_CHUNK, W), jnp.float32) for _ in range(_NBUF)],
            [pltpu.SemaphoreType.DMA for _ in range(_NBUF)],
            [pltpu.SemaphoreType.DMA for _ in range(_NBUF)],
            pltpu.VMEM((_HP, W), jnp.float32),
        ],
    )(_sc_body)
    out = run(xr, jnp.asarray(_MASK_ROWS))
    return out.reshape(B, C, H, W)


kernel = _kernel_sc


# speedup vs baseline: 2.0532x; 2.0532x over previous
import functools

import numpy as np
import jax
import jax.numpy as jnp
from jax import lax
from jax.experimental import pallas as pl
from jax.experimental.pallas import tpu as pltpu
from jax.experimental.pallas import tpu_sc as plsc

_PS = 16
_H = 512
_W = 512
_HP = _H // _PS
_WP = _W // _PS
_TOTAL = _HP * _WP
_NUM_MASK = int(0.75 * _TOTAL)
_MASK_VALUE = 0.5
_BLOCK_ROWS = 4096

_MASK_BITS_HEX = (
    "bfbe67fd4f3fa775bcfdfe7dffefe7bbf0f9ff37fadbfefe6c7bfffaff4b5b6f"
    "fdabf03bd7ffbd7ffdeffa7f5bbe7fefe8e74efffffff7feeefffbf7f5f3b57d"
    "f9baefd79ff8febdf7f1affaceed6bb4fdcfdc3e677fbcbb4fbbf4cad97fb7ef"
    "efffffd49e3ecffdff9fe299ff5b5e9f0a65d66b75effbeefd76bdefe3dfeffd"
)


def _full_mask() -> np.ndarray:
    val = int(_MASK_BITS_HEX, 16)
    patch_mask = np.array([(val >> i) & 1 for i in range(_TOTAL)], dtype=bool)
    grid2d = patch_mask.reshape(_HP, _WP)
    return np.repeat(np.repeat(grid2d, _PS, axis=0), _PS, axis=1)


_MASK_BLOCK = np.tile(_full_mask(), (_BLOCK_ROWS // _H, 1)).astype(np.float32)


def _select_body(m_ref, x_ref, o_ref):
    o_ref[...] = jnp.where(m_ref[...] != 0.0, _MASK_VALUE, x_ref[...])


def _kernel_tc(x):
    B, C, H, W = x.shape
    rows = B * C * H
    xr = x.reshape(rows, W)
    mask = jnp.asarray(_MASK_BLOCK)
    out = pl.pallas_call(
        _select_body,
        grid=(rows // _BLOCK_ROWS,),
        in_specs=[
            pl.BlockSpec((_BLOCK_ROWS, W), lambda i: (0, 0)),
            pl.BlockSpec((_BLOCK_ROWS, W), lambda i: (i, 0)),
        ],
        out_specs=pl.BlockSpec((_BLOCK_ROWS, W), lambda i: (i, 0)),
        out_shape=jax.ShapeDtypeStruct((rows, W), x.dtype),
        compiler_params=pltpu.CompilerParams(
            dimension_semantics=("parallel",),
        ),
    )(mask, xr)
    return out.reshape(B, C, H, W)


_L = 16
_NW = 32
_ROWS = 64 * 3 * _H
_RPW = _ROWS // _NW
_CHUNK = 2 * _PS
_NCHUNK = _RPW // _CHUNK
_NBUF = 2


def _mask_row_table() -> np.ndarray:
    full = _full_mask()
    return full[::_PS, :].astype(np.float32)


_MASK_ROWS = _mask_row_table()


def _sc_body(x_hbm, mrow_hbm, out_hbm, ibufs, obufs, isems, osems, mtab):
    wid = lax.axis_index("s") * 2 + lax.axis_index("c")
    base = wid * _NCHUNK
    pltpu.sync_copy(mrow_hbm, mtab)

    def in_slice(ci):
        return x_hbm.at[pl.ds((base + ci) * _CHUNK, _CHUNK)]

    def out_slice(ci):
        return out_hbm.at[pl.ds((base + ci) * _CHUNK, _CHUNK)]

    for b in range(_NBUF):
        pltpu.async_copy(in_slice(b), ibufs[b], isems[b])

    fill = jnp.full((_L,), _MASK_VALUE, jnp.float32)

    def round_body(gp, carry):
        for b in range(_NBUF):
            ci = gp * _NBUF + b
            pltpu.make_async_copy(in_slice(ci), ibufs[b], isems[b]).wait()
            for half in range(2):
                pr = lax.rem((base + ci) * 2 + half, _HP)
                for c in range(_W // _L):
                    sl = pl.ds(c * _L, _L)
                    mval = mtab[pr, sl]
                    for r in range(_PS):
                        rr = half * _PS + r
                        obufs[b][rr, sl] = jnp.where(
                            mval != 0.0, fill, ibufs[b][rr, sl])
            @pl.when(gp > 0)
            def _():
                pltpu.make_async_copy(
                    obufs[b], out_slice(ci - _NBUF), osems[b]).wait()
            pltpu.async_copy(obufs[b], out_slice(ci), osems[b])
            @pl.when(ci + _NBUF < _NCHUNK)
            def _():
                pltpu.async_copy(in_slice(ci + _NBUF), ibufs[b], isems[b])
        return carry

    lax.fori_loop(0, _NCHUNK // _NBUF, round_body, 0)

    for b in range(_NBUF):
        pltpu.make_async_copy(
            obufs[b], out_slice(_NCHUNK - _NBUF + b), osems[b]).wait()


def _kernel_sc(x):
    B, C, H, W = x.shape
    xr = x.reshape(B * C * H, W)
    mesh = plsc.VectorSubcoreMesh(core_axis_name="c", subcore_axis_name="s")
    run = functools.partial(
        pl.kernel,
        mesh=mesh,
        out_type=jax.ShapeDtypeStruct((B * C * H, W), x.dtype),
        scratch_types=[
            [pltpu.VMEM((_CHUNK, W), jnp.float32) for _ in range(_NBUF)],
            [pltpu.VMEM((_CHUNK, W), jnp.float32) for _ in range(_NBUF)],
            [pltpu.SemaphoreType.DMA for _ in range(_NBUF)],
            [pltpu.SemaphoreType.DMA for _ in range(_NBUF)],
            pltpu.VMEM((_HP, W), jnp.float32),
        ],
    )(_sc_body)
    out = run(xr, jnp.asarray(_MASK_ROWS))
    return out.reshape(B, C, H, W)


kernel = _kernel_tc
